# Initial kernel scaffold; baseline (speedup 1.0000x reference)
import jax
import jax.numpy as jnp
from jax.experimental import pallas as pl


def kernel(x, edge_index, edge_attr, W_node, b_node, W_edge, b_edge, gamma, beta, fc_W, fc_b, attn_W, attn_b):
    raise NotImplementedError("write your pallas kernel here")


# speedup vs baseline: 2.1336x; 2.1336x over previous
import functools

import jax
import jax.numpy as jnp
from jax import lax
from jax.experimental import pallas as pl
from jax.experimental.pallas import tpu as pltpu
from jax.experimental.pallas import tpu_sc as plsc

N = 10000
E = 320000
D = 128
DE = 16
H = 64

NC = 2
NS = 16
CHUNK = 64
EPT = 20480
E_PAD = EPT * NS
NSTEP = EPT // CHUNK
N_ACC = 10240
ROWS_PER_TILE = N_ACC // NS
FLUSH_R = 64

F32 = jnp.float32


_RB = 2000
_NB = N // _RB


def _k1_body(x_ref, w_ref, b_ref, h_ref, s_ref, ss_ref):
  h = jnp.dot(x_ref[...], w_ref[...], preferred_element_type=F32) + b_ref[...]
  h_ref[...] = h
  s_ref[...] = jnp.sum(h, axis=0, keepdims=True)[None]
  ss_ref[...] = jnp.sum(h * h, axis=0, keepdims=True)[None]


_k1 = pl.pallas_call(
    _k1_body,
    grid=(_NB,),
    in_specs=[
        pl.BlockSpec((_RB, D), lambda i: (i, 0)),
        pl.BlockSpec((D, D), lambda i: (0, 0)),
        pl.BlockSpec((1, D), lambda i: (0, 0)),
    ],
    out_specs=[
        pl.BlockSpec((_RB, D), lambda i: (i, 0)),
        pl.BlockSpec((1, 1, D), lambda i: (i, 0, 0)),
        pl.BlockSpec((1, 1, D), lambda i: (i, 0, 0)),
    ],
    out_shape=[
        jax.ShapeDtypeStruct((N, D), F32),
        jax.ShapeDtypeStruct((_NB, 1, D), F32),
        jax.ShapeDtypeStruct((_NB, 1, D), F32),
    ],
)


def _k2_first_body(h_ref, s_ref, ss_ref, g_ref, be_ref, fw_ref, fb_ref,
                   fwh_ref, fbh_ref, wd_ref, wsh_ref, src2_ref, pd_ref):
  mean = jnp.sum(s_ref[...], axis=0) / N
  var = jnp.sum(ss_ref[...], axis=0) / N - mean * mean
  hn = (h_ref[...] - mean) * (g_ref[...] * lax.rsqrt(var + 1e-5)) + be_ref[...]
  t = jnp.dot(hn, fw_ref[...], preferred_element_type=F32) + fb_ref[...]
  ps_h = jnp.dot(t, wsh_ref[0], preferred_element_type=F32)
  t_h = jnp.dot(hn, fwh_ref[0], preferred_element_type=F32) + fbh_ref[0]
  src2_ref[0] = jnp.concatenate([ps_h, t_h], axis=1)
  pd_ref[...] = jnp.dot(t, wd_ref[...], preferred_element_type=F32)


_k2_first = pl.pallas_call(
    _k2_first_body,
    grid=(_NB, NC),
    in_specs=[
        pl.BlockSpec((_RB, D), lambda i, c: (i, 0)),
        pl.BlockSpec((_NB, 1, D), lambda i, c: (0, 0, 0)),
        pl.BlockSpec((_NB, 1, D), lambda i, c: (0, 0, 0)),
        pl.BlockSpec((1, D), lambda i, c: (0, 0)),
        pl.BlockSpec((1, D), lambda i, c: (0, 0)),
        pl.BlockSpec((D, D), lambda i, c: (0, 0)),
        pl.BlockSpec((1, D), lambda i, c: (0, 0)),
        pl.BlockSpec((1, D, H), lambda i, c: (c, 0, 0)),
        pl.BlockSpec((1, 1, H), lambda i, c: (c, 0, 0)),
        pl.BlockSpec((D, D), lambda i, c: (0, 0)),
        pl.BlockSpec((1, D, H), lambda i, c: (c, 0, 0)),
    ],
    out_specs=[
        pl.BlockSpec((1, _RB, D), lambda i, c: (c, i, 0)),
        pl.BlockSpec((_RB, D), lambda i, c: (i, 0)),
    ],
    out_shape=[
        jax.ShapeDtypeStruct((NC, N, D), F32),
        jax.ShapeDtypeStruct((N, D), F32),
    ],
)


def _k2_mid_body(acc_ref, fw_ref, fb_ref, fwh_ref, fbh_ref, wd_ref, wsh_ref,
                 src2_ref, pd_ref):
  a0, a1 = acc_ref[0], acc_ref[1]
  h = jnp.concatenate(
      [a0[:, H:] / (a0[:, :H] + 1e-16), a1[:, H:] / (a1[:, :H] + 1e-16)],
      axis=1)
  t = jnp.dot(h, fw_ref[...], preferred_element_type=F32) + fb_ref[...]
  ps_h = jnp.dot(t, wsh_ref[0], preferred_element_type=F32)
  t_h = jnp.dot(h, fwh_ref[0], preferred_element_type=F32) + fbh_ref[0]
  src2_ref[0] = jnp.concatenate([ps_h, t_h], axis=1)
  pd_ref[...] = jnp.dot(t, wd_ref[...], preferred_element_type=F32)


_k2_mid = pl.pallas_call(
    _k2_mid_body,
    grid=(_NB, NC),
    in_specs=[
        pl.BlockSpec((NC, _RB, D), lambda i, c: (0, i, 0)),
        pl.BlockSpec((D, D), lambda i, c: (0, 0)),
        pl.BlockSpec((1, D), lambda i, c: (0, 0)),
        pl.BlockSpec((1, D, H), lambda i, c: (c, 0, 0)),
        pl.BlockSpec((1, 1, H), lambda i, c: (c, 0, 0)),
        pl.BlockSpec((D, D), lambda i, c: (0, 0)),
        pl.BlockSpec((1, D, H), lambda i, c: (c, 0, 0)),
    ],
    out_specs=[
        pl.BlockSpec((1, _RB, D), lambda i, c: (c, i, 0)),
        pl.BlockSpec((_RB, D), lambda i, c: (i, 0)),
    ],
    out_shape=[
        jax.ShapeDtypeStruct((NC, N, D), F32),
        jax.ShapeDtypeStruct((N, D), F32),
    ],
)


_EB = 2048
_NEB = E_PAD // _EB


def _k3_body(ea_ref, we_ref, be_ref, wa_ref, ab_ref, pe_ref):
  i = pl.program_id(0)
  m = jnp.dot(we_ref[...], wa_ref[...], preferred_element_type=F32)
  cvec = (jnp.dot(be_ref[...], wa_ref[...], preferred_element_type=F32)
          + ab_ref[...])
  pe = jnp.dot(ea_ref[...], m, preferred_element_type=F32) + cvec
  row = i * _EB + lax.broadcasted_iota(jnp.int32, (_EB, 1), 0)
  pe_ref[...] = jnp.where(row < E, pe, -1e30)


_k3 = pl.pallas_call(
    _k3_body,
    grid=(_NEB,),
    in_specs=[
        pl.BlockSpec((_EB, DE), lambda i: (i, 0)),
        pl.BlockSpec((DE, DE), lambda i: (0, 0)),
        pl.BlockSpec((1, DE), lambda i: (0, 0)),
        pl.BlockSpec((DE, D), lambda i: (0, 0)),
        pl.BlockSpec((1, D), lambda i: (0, 0)),
    ],
    out_specs=pl.BlockSpec((_EB, D), lambda i: (i, 0)),
    out_shape=jax.ShapeDtypeStruct((E_PAD, D), F32),
)


def _k4_body(acc_ref, out_ref):
  a0, a1 = acc_ref[0], acc_ref[1]
  out_ref[...] = jnp.concatenate(
      [a0[:, H:] / (a0[:, :H] + 1e-16), a1[:, H:] / (a1[:, :H] + 1e-16)],
      axis=1)


_k4 = pl.pallas_call(
    _k4_body,
    grid=(_NB,),
    in_specs=[
        pl.BlockSpec((NC, _RB, D), lambda i: (0, i, 0)),
    ],
    out_specs=pl.BlockSpec((_RB, D), lambda i: (i, 0)),
    out_shape=jax.ShapeDtypeStruct((N, D), F32),
)


@functools.cache
def _build_sc_edge_pass():
  sc_mesh = plsc.VectorSubcoreMesh(
      core_axis_name="c", subcore_axis_name="s", num_cores=NC,
      num_subcores=NS)

  @functools.partial(
      pl.kernel,
      out_type=jax.ShapeDtypeStruct((NC * N_ACC, D), F32),
      mesh=sc_mesh,
      scratch_types=[
          pltpu.VMEM((CHUNK,), jnp.int32),
          pltpu.VMEM((CHUNK,), jnp.int32),
          pltpu.VMEM((CHUNK, D), F32),
          pltpu.VMEM((CHUNK, D), F32),
          pltpu.VMEM((CHUNK, D), F32),
          pltpu.VMEM((CHUNK, D), F32),
          pltpu.VMEM_SHARED((N_ACC, D), F32),
          pltpu.SemaphoreType.DMA,
          pltpu.SemaphoreType.DMA,
      ],
  )
  def sc_edge_pass(src_hbm, dst_hbm, src2_hbm, pd_hbm, pe_hbm,
                   acc_hbm,
                   idx_src, idx_dst, src_rows, pd_rows, pe_rows, part,
                   acc, sem0, sem1):
    c = lax.axis_index("c")
    s = lax.axis_index("s")
    cH = c * H

    zero = jnp.zeros((16,), F32)
    for r in range(FLUSH_R):
      for j in range(D // 16):
        pe_rows[r, pl.ds(j * 16, 16)] = zero
    for k in range(ROWS_PER_TILE // FLUSH_R):
      rows = pl.ds(s * ROWS_PER_TILE + k * FLUSH_R, FLUSH_R)
      pltpu.sync_copy(pe_rows, acc.at[rows])
    plsc.subcore_barrier()

    def step(g, carry):
      base = s * EPT + g * CHUNK
      pltpu.sync_copy(src_hbm.at[pl.ds(base, CHUNK)], idx_src)
      pltpu.sync_copy(dst_hbm.at[pl.ds(base, CHUNK)], idx_dst)
      for j in range(CHUNK // 16):
        sl = pl.ds(j * 16, 16)
        idx_src[sl] = idx_src[sl] + c * N
      cp0 = pltpu.async_copy(src2_hbm.at[idx_src], src_rows, sem0)
      cp1 = pltpu.async_copy(pd_hbm.at[idx_dst], pd_rows, sem1)
      pltpu.sync_copy(pe_hbm.at[pl.ds(base, CHUNK)], pe_rows)
      cp0.wait()
      cp1.wait()

      def edge_body(e, carry2):
        for j in range(H // 16):
          slh = pl.ds(cH + j * 16, 16)
          sl = pl.ds(j * 16, 16)
          a = pd_rows[e, slh] + src_rows[e, sl] + pe_rows[e, slh]
          a = jnp.maximum(a, a * 0.2)
          w = jnp.exp(a)
          part[e, sl] = w
          part[e, pl.ds(H + j * 16, 16)] = w * src_rows[e, pl.ds(H + j * 16, 16)]
        return carry2

      lax.fori_loop(0, CHUNK, edge_body, 0)
      pltpu.sync_copy(part, acc.at[idx_dst], add=True)
      return carry

    lax.fori_loop(0, NSTEP, step, 0)
    plsc.subcore_barrier()

    for k in range(ROWS_PER_TILE // FLUSH_R):
      r0 = s * ROWS_PER_TILE + k * FLUSH_R
      pltpu.sync_copy(acc.at[pl.ds(r0, FLUSH_R)], pe_rows)
      pltpu.sync_copy(pe_rows, acc_hbm.at[pl.ds(c * N_ACC + r0, FLUSH_R)])

  return sc_edge_pass


def _split_cols(w):
  return w.reshape(D, NC, H).transpose(1, 0, 2)


def kernel(x, edge_index, edge_attr, W_node, b_node, W_edge, b_edge,
           gamma, beta, fc_W, fc_b, attn_W, attn_b):
  src = edge_index[0].astype(jnp.int32)
  dst = edge_index[1].astype(jnp.int32)
  src_pad = jnp.pad(src, (0, E_PAD - E))
  dst_pad = jnp.pad(dst, (0, E_PAD - E))
  ea_pad = jnp.pad(edge_attr.astype(F32), ((0, E_PAD - E), (0, 0)))

  h_pre, psum, psumsq = _k1(x, W_node, b_node.reshape(1, D))

  acc = None
  for l in range(3):
    aW = attn_W[l]
    wd, ws, wa = aW[:D], aW[D:2 * D], aW[2 * D:]
    fw = fc_W[l]
    fwh = _split_cols(fw)
    fbh = fc_b[l].reshape(NC, 1, H)
    wsh = _split_cols(ws)
    if l == 0:
      src2, pd = _k2_first(h_pre, psum, psumsq, gamma.reshape(1, D),
                           beta.reshape(1, D), fw, fc_b[0].reshape(1, D),
                           fwh, fbh, wd, wsh)
    else:
      src2, pd = _k2_mid(acc, fw, fc_b[l].reshape(1, D), fwh, fbh, wd, wsh)
    pe = _k3(ea_pad, W_edge, b_edge.reshape(1, DE), wa,
             attn_b[l].reshape(1, D))
    acc = _build_sc_edge_pass()(
        src_pad, dst_pad, src2.reshape(NC * N, D), pd, pe)
    acc = acc.reshape(NC, N_ACC, D)[:, :N]

  h = _k4(acc)
  return h.reshape(1, N, D)


# speedup vs baseline: 2.9284x; 1.3725x over previous
import functools

import jax
import jax.numpy as jnp
from jax import lax
from jax.experimental import pallas as pl
from jax.experimental.pallas import tpu as pltpu
from jax.experimental.pallas import tpu_sc as plsc

N = 10000
E = 320000
D = 128
DE = 16
H = 64

NC = 2
NS = 16
CHUNK = 64
EPT = 20480
E_PAD = EPT * NS
NSTEP = EPT // CHUNK
N_ACC = 10112
ROWS_PER_TILE = N_ACC // NS

F32 = jnp.float32


_RB = 2000
_NB = N // _RB


def _k1_body(x_ref, w_ref, b_ref, h_ref, s_ref, ss_ref):
  h = jnp.dot(x_ref[...], w_ref[...], preferred_element_type=F32) + b_ref[...]
  h_ref[...] = h
  s_ref[...] = jnp.sum(h, axis=0, keepdims=True)[None]
  ss_ref[...] = jnp.sum(h * h, axis=0, keepdims=True)[None]


_k1 = pl.pallas_call(
    _k1_body,
    grid=(_NB,),
    in_specs=[
        pl.BlockSpec((_RB, D), lambda i: (i, 0)),
        pl.BlockSpec((D, D), lambda i: (0, 0)),
        pl.BlockSpec((1, D), lambda i: (0, 0)),
    ],
    out_specs=[
        pl.BlockSpec((_RB, D), lambda i: (i, 0)),
        pl.BlockSpec((1, 1, D), lambda i: (i, 0, 0)),
        pl.BlockSpec((1, 1, D), lambda i: (i, 0, 0)),
    ],
    out_shape=[
        jax.ShapeDtypeStruct((N, D), F32),
        jax.ShapeDtypeStruct((_NB, 1, D), F32),
        jax.ShapeDtypeStruct((_NB, 1, D), F32),
    ],
)


def _k2_first_body(h_ref, s_ref, ss_ref, g_ref, be_ref, fw_ref, fb_ref,
                   fwh_ref, fbh_ref, wd_ref, wsh_ref, src2_ref, pd_ref):
  mean = jnp.sum(s_ref[...], axis=0) / N
  var = jnp.sum(ss_ref[...], axis=0) / N - mean * mean
  hn = (h_ref[...] - mean) * (g_ref[...] * lax.rsqrt(var + 1e-5)) + be_ref[...]
  t = jnp.dot(hn, fw_ref[...], preferred_element_type=F32) + fb_ref[...]
  ps_h = jnp.dot(t, wsh_ref[0], preferred_element_type=F32)
  t_h = jnp.dot(hn, fwh_ref[0], preferred_element_type=F32) + fbh_ref[0]
  src2_ref[0] = jnp.concatenate([ps_h, t_h], axis=1)
  pd_ref[...] = jnp.dot(t, wd_ref[...], preferred_element_type=F32)


_k2_first = pl.pallas_call(
    _k2_first_body,
    grid=(_NB, NC),
    in_specs=[
        pl.BlockSpec((_RB, D), lambda i, c: (i, 0)),
        pl.BlockSpec((_NB, 1, D), lambda i, c: (0, 0, 0)),
        pl.BlockSpec((_NB, 1, D), lambda i, c: (0, 0, 0)),
        pl.BlockSpec((1, D), lambda i, c: (0, 0)),
        pl.BlockSpec((1, D), lambda i, c: (0, 0)),
        pl.BlockSpec((D, D), lambda i, c: (0, 0)),
        pl.BlockSpec((1, D), lambda i, c: (0, 0)),
        pl.BlockSpec((1, D, H), lambda i, c: (c, 0, 0)),
        pl.BlockSpec((1, 1, H), lambda i, c: (c, 0, 0)),
        pl.BlockSpec((D, D), lambda i, c: (0, 0)),
        pl.BlockSpec((1, D, H), lambda i, c: (c, 0, 0)),
    ],
    out_specs=[
        pl.BlockSpec((1, _RB, D), lambda i, c: (c, i, 0)),
        pl.BlockSpec((_RB, D), lambda i, c: (i, 0)),
    ],
    out_shape=[
        jax.ShapeDtypeStruct((NC, N, D), F32),
        jax.ShapeDtypeStruct((N, D), F32),
    ],
)


def _k2_mid_body(acc_ref, fw_ref, fb_ref, fwh_ref, fbh_ref, wd_ref, wsh_ref,
                 src2_ref, pd_ref):
  a0, a1 = acc_ref[0], acc_ref[1]
  h = jnp.concatenate(
      [a0[:, H:] / (a0[:, :H] + 1e-16), a1[:, H:] / (a1[:, :H] + 1e-16)],
      axis=1)
  t = jnp.dot(h, fw_ref[...], preferred_element_type=F32) + fb_ref[...]
  ps_h = jnp.dot(t, wsh_ref[0], preferred_element_type=F32)
  t_h = jnp.dot(h, fwh_ref[0], preferred_element_type=F32) + fbh_ref[0]
  src2_ref[0] = jnp.concatenate([ps_h, t_h], axis=1)
  pd_ref[...] = jnp.dot(t, wd_ref[...], preferred_element_type=F32)


_k2_mid = pl.pallas_call(
    _k2_mid_body,
    grid=(_NB, NC),
    in_specs=[
        pl.BlockSpec((NC, _RB, D), lambda i, c: (0, i, 0)),
        pl.BlockSpec((D, D), lambda i, c: (0, 0)),
        pl.BlockSpec((1, D), lambda i, c: (0, 0)),
        pl.BlockSpec((1, D, H), lambda i, c: (c, 0, 0)),
        pl.BlockSpec((1, 1, H), lambda i, c: (c, 0, 0)),
        pl.BlockSpec((D, D), lambda i, c: (0, 0)),
        pl.BlockSpec((1, D, H), lambda i, c: (c, 0, 0)),
    ],
    out_specs=[
        pl.BlockSpec((1, _RB, D), lambda i, c: (c, i, 0)),
        pl.BlockSpec((_RB, D), lambda i, c: (i, 0)),
    ],
    out_shape=[
        jax.ShapeDtypeStruct((NC, N, D), F32),
        jax.ShapeDtypeStruct((N, D), F32),
    ],
)


_EB = 2048
_NEB = E_PAD // _EB


def _k3_body(ea_ref, we_ref, be_ref, wa_ref, ab_ref, pe_ref):
  i = pl.program_id(0)
  m = jnp.dot(we_ref[...], wa_ref[...], preferred_element_type=F32)
  cvec = (jnp.dot(be_ref[...], wa_ref[...], preferred_element_type=F32)
          + ab_ref[...])
  pe = jnp.dot(ea_ref[...], m, preferred_element_type=F32) + cvec
  row = i * _EB + lax.broadcasted_iota(jnp.int32, (_EB, 1), 0)
  pe_ref[...] = jnp.where(row < E, pe, -1e30)


_k3 = pl.pallas_call(
    _k3_body,
    grid=(_NEB,),
    in_specs=[
        pl.BlockSpec((_EB, DE), lambda i: (i, 0)),
        pl.BlockSpec((DE, DE), lambda i: (0, 0)),
        pl.BlockSpec((1, DE), lambda i: (0, 0)),
        pl.BlockSpec((DE, D), lambda i: (0, 0)),
        pl.BlockSpec((1, D), lambda i: (0, 0)),
    ],
    out_specs=pl.BlockSpec((_EB, D), lambda i: (i, 0)),
    out_shape=jax.ShapeDtypeStruct((E_PAD, D), F32),
)


def _k4_body(acc_ref, out_ref):
  a0, a1 = acc_ref[0], acc_ref[1]
  out_ref[...] = jnp.concatenate(
      [a0[:, H:] / (a0[:, :H] + 1e-16), a1[:, H:] / (a1[:, :H] + 1e-16)],
      axis=1)


_k4 = pl.pallas_call(
    _k4_body,
    grid=(_NB,),
    in_specs=[
        pl.BlockSpec((NC, _RB, D), lambda i: (0, i, 0)),
    ],
    out_specs=pl.BlockSpec((_RB, D), lambda i: (i, 0)),
    out_shape=jax.ShapeDtypeStruct((N, D), F32),
)


@functools.cache
def _build_sc_edge_pass():
  sc_mesh = plsc.VectorSubcoreMesh(
      core_axis_name="c", subcore_axis_name="s", num_cores=NC,
      num_subcores=NS)

  @functools.partial(
      pl.kernel,
      out_type=jax.ShapeDtypeStruct((NC * N_ACC, D), F32),
      mesh=sc_mesh,
      scratch_types=[
          pltpu.VMEM((CHUNK,), jnp.int32),
          pltpu.VMEM((CHUNK,), jnp.int32),
          pltpu.VMEM((CHUNK,), jnp.int32),
          pltpu.VMEM((CHUNK,), jnp.int32),
          pltpu.VMEM((CHUNK, D), F32),
          pltpu.VMEM((CHUNK, D), F32),
          pltpu.VMEM((CHUNK, D), F32),
          pltpu.VMEM((CHUNK, D), F32),
          pltpu.VMEM((CHUNK, D), F32),
          pltpu.VMEM((CHUNK, D), F32),
          pltpu.VMEM_SHARED((N_ACC, D), F32),
          pltpu.SemaphoreType.DMA,
          pltpu.SemaphoreType.DMA,
      ],
  )
  def sc_edge_pass(src_hbm, dst_hbm, src2_hbm, pd_hbm, pe_hbm,
                   acc_hbm,
                   isa, ida, isb, idb, sra, pda, pea, srb, pdb, peb,
                   acc, sema, semb):
    c = lax.axis_index("c")
    s = lax.axis_index("s")
    cH = c * H
    set_a = (isa, ida, sra, pda, pea, sema)
    set_b = (isb, idb, srb, pdb, peb, semb)

    zero = jnp.zeros((16,), F32)
    for r in range(8):
      for j in range(D // 16):
        sra[r, pl.ds(j * 16, 16)] = zero

    def zinit(k, carry):
      pltpu.sync_copy(sra.at[pl.ds(0, 8)],
                      acc.at[pl.ds(s * ROWS_PER_TILE + k * 8, 8)])
      return carry

    lax.fori_loop(0, ROWS_PER_TILE // 8, zinit, 0)
    plsc.subcore_barrier()

    def prefetch(g, bufs):
      i_s, i_d, sr, pd_, pe_, sem = bufs
      base = s * EPT + g * CHUNK
      pltpu.sync_copy(src_hbm.at[pl.ds(base, CHUNK)], i_s)
      pltpu.sync_copy(dst_hbm.at[pl.ds(base, CHUNK)], i_d)
      for j in range(CHUNK // 16):
        sl = pl.ds(j * 16, 16)
        i_s[sl] = i_s[sl] + c * N
      pltpu.async_copy(src2_hbm.at[i_s], sr, sem)
      pltpu.async_copy(pd_hbm.at[i_d], pd_, sem)
      pltpu.async_copy(pe_hbm.at[pl.ds(base, CHUNK)], pe_, sem)

    def wait_set(bufs):
      i_s, i_d, sr, pd_, pe_, sem = bufs
      pltpu.make_async_copy(src2_hbm.at[i_s], sr, sem).wait()
      pltpu.make_async_copy(pd_hbm.at[i_d], pd_, sem).wait()
      pltpu.make_async_copy(pe_hbm.at[pl.ds(0, CHUNK)], pe_, sem).wait()

    def compute_scatter(bufs):
      i_s, i_d, sr, pd_, pe_, sem = bufs

      def edge_body(e, carry2):
        for j in range(H // 16):
          slh = pl.ds(cH + j * 16, 16)
          sl = pl.ds(j * 16, 16)
          slt = pl.ds(H + j * 16, 16)
          a = pd_[e, slh] + sr[e, sl] + pe_[e, slh]
          a = jnp.maximum(a, a * 0.2)
          w = jnp.exp(a)
          tv = sr[e, slt]
          pd_[e, sl] = w
          pd_[e, slt] = w * tv
        return carry2

      lax.fori_loop(0, CHUNK, edge_body, 0)
      pltpu.sync_copy(pd_, acc.at[i_d], add=True)

    def pair_body(i, carry):
      g0 = 2 * i
      wait_set(set_a)
      prefetch(g0 + 1, set_b)
      compute_scatter(set_a)
      wait_set(set_b)

      @pl.when(i + 1 < NSTEP // 2)
      def _():
        prefetch(g0 + 2, set_a)

      compute_scatter(set_b)
      return carry

    prefetch(0, set_a)
    lax.fori_loop(0, NSTEP // 2, pair_body, 0)
    plsc.subcore_barrier()

    def flush(k, carry):
      r0 = s * ROWS_PER_TILE + k * 64
      pltpu.sync_copy(acc.at[pl.ds(r0, 64)], peb)
      pltpu.sync_copy(peb, acc_hbm.at[pl.ds(c * N_ACC + r0, 64)])
      return carry

    lax.fori_loop(0, ROWS_PER_TILE // 64, flush, 0)
    tail = ROWS_PER_TILE % 64
    if tail:
      r0 = s * ROWS_PER_TILE + (ROWS_PER_TILE // 64) * 64
      pltpu.sync_copy(acc.at[pl.ds(r0, tail)], peb.at[pl.ds(0, tail)])
      pltpu.sync_copy(peb.at[pl.ds(0, tail)],
                      acc_hbm.at[pl.ds(c * N_ACC + r0, tail)])

  return sc_edge_pass


def _split_cols(w):
  return w.reshape(D, NC, H).transpose(1, 0, 2)


def kernel(x, edge_index, edge_attr, W_node, b_node, W_edge, b_edge,
           gamma, beta, fc_W, fc_b, attn_W, attn_b):
  src = edge_index[0].astype(jnp.int32)
  dst = edge_index[1].astype(jnp.int32)
  src_pad = jnp.pad(src, (0, E_PAD - E))
  dst_pad = jnp.pad(dst, (0, E_PAD - E))
  ea_pad = jnp.pad(edge_attr.astype(F32), ((0, E_PAD - E), (0, 0)))

  h_pre, psum, psumsq = _k1(x, W_node, b_node.reshape(1, D))

  acc = None
  for l in range(3):
    aW = attn_W[l]
    wd, ws, wa = aW[:D], aW[D:2 * D], aW[2 * D:]
    fw = fc_W[l]
    fwh = _split_cols(fw)
    fbh = fc_b[l].reshape(NC, 1, H)
    wsh = _split_cols(ws)
    if l == 0:
      src2, pd = _k2_first(h_pre, psum, psumsq, gamma.reshape(1, D),
                           beta.reshape(1, D), fw, fc_b[0].reshape(1, D),
                           fwh, fbh, wd, wsh)
    else:
      src2, pd = _k2_mid(acc, fw, fc_b[l].reshape(1, D), fwh, fbh, wd, wsh)
    pe = _k3(ea_pad, W_edge, b_edge.reshape(1, DE), wa,
             attn_b[l].reshape(1, D))
    acc = _build_sc_edge_pass()(
        src_pad, dst_pad, src2.reshape(NC * N, D), pd, pe)
    acc = acc.reshape(NC, N_ACC, D)[:, :N]

  h = _k4(acc)
  return h.reshape(1, N, D)


# speedup vs baseline: 4.0369x; 1.3785x over previous
import functools

import jax
import jax.numpy as jnp
from jax import lax
from jax.experimental import pallas as pl
from jax.experimental.pallas import tpu as pltpu
from jax.experimental.pallas import tpu_sc as plsc

N = 10000
E = 320000
D = 128
DE = 16
H = 64

NC = 2
NS = 16
CHUNK = 64
EPT = 20480
E_PAD = EPT * NS
NSTEP = EPT // CHUNK
N_ACC = 10112
ROWS_PER_TILE = N_ACC // NS

F32 = jnp.float32


_RB = 2000
_NB = N // _RB


def _k1_body(x_ref, w_ref, b_ref, h_ref, s_ref, ss_ref):
  h = jnp.dot(x_ref[...], w_ref[...], preferred_element_type=F32) + b_ref[...]
  h_ref[...] = h
  s_ref[...] = jnp.sum(h, axis=0, keepdims=True)[None]
  ss_ref[...] = jnp.sum(h * h, axis=0, keepdims=True)[None]


_k1 = pl.pallas_call(
    _k1_body,
    grid=(_NB,),
    in_specs=[
        pl.BlockSpec((_RB, D), lambda i: (i, 0)),
        pl.BlockSpec((D, D), lambda i: (0, 0)),
        pl.BlockSpec((1, D), lambda i: (0, 0)),
    ],
    out_specs=[
        pl.BlockSpec((_RB, D), lambda i: (i, 0)),
        pl.BlockSpec((1, 1, D), lambda i: (i, 0, 0)),
        pl.BlockSpec((1, 1, D), lambda i: (i, 0, 0)),
    ],
    out_shape=[
        jax.ShapeDtypeStruct((N, D), F32),
        jax.ShapeDtypeStruct((_NB, 1, D), F32),
        jax.ShapeDtypeStruct((_NB, 1, D), F32),
    ],
)


def _k2_first_body(h_ref, s_ref, ss_ref, g_ref, be_ref, fw_ref, fb_ref,
                   fwh_ref, fbh_ref, wd_ref, wsh_ref, src2_ref, pd_ref):
  mean = jnp.sum(s_ref[...], axis=0) / N
  var = jnp.sum(ss_ref[...], axis=0) / N - mean * mean
  hn = (h_ref[...] - mean) * (g_ref[...] * lax.rsqrt(var + 1e-5)) + be_ref[...]
  t = jnp.dot(hn, fw_ref[...], preferred_element_type=F32) + fb_ref[...]
  ps_h = jnp.dot(t, wsh_ref[0], preferred_element_type=F32)
  t_h = jnp.dot(hn, fwh_ref[0], preferred_element_type=F32) + fbh_ref[0]
  src2_ref[0] = jnp.concatenate([ps_h, t_h], axis=1)
  pd_ref[...] = jnp.dot(t, wd_ref[...], preferred_element_type=F32)


_k2_first = pl.pallas_call(
    _k2_first_body,
    grid=(_NB, NC),
    in_specs=[
        pl.BlockSpec((_RB, D), lambda i, c: (i, 0)),
        pl.BlockSpec((_NB, 1, D), lambda i, c: (0, 0, 0)),
        pl.BlockSpec((_NB, 1, D), lambda i, c: (0, 0, 0)),
        pl.BlockSpec((1, D), lambda i, c: (0, 0)),
        pl.BlockSpec((1, D), lambda i, c: (0, 0)),
        pl.BlockSpec((D, D), lambda i, c: (0, 0)),
        pl.BlockSpec((1, D), lambda i, c: (0, 0)),
        pl.BlockSpec((1, D, H), lambda i, c: (c, 0, 0)),
        pl.BlockSpec((1, 1, H), lambda i, c: (c, 0, 0)),
        pl.BlockSpec((D, D), lambda i, c: (0, 0)),
        pl.BlockSpec((1, D, H), lambda i, c: (c, 0, 0)),
    ],
    out_specs=[
        pl.BlockSpec((1, _RB, D), lambda i, c: (c, i, 0)),
        pl.BlockSpec((_RB, D), lambda i, c: (i, 0)),
    ],
    out_shape=[
        jax.ShapeDtypeStruct((NC, N, D), F32),
        jax.ShapeDtypeStruct((N, D), F32),
    ],
)


def _k2_mid_body(acc_ref, fw_ref, fb_ref, fwh_ref, fbh_ref, wd_ref, wsh_ref,
                 src2_ref, pd_ref):
  a0, a1 = acc_ref[0], acc_ref[1]
  h = jnp.concatenate(
      [a0[:, H:] / (a0[:, :H] + 1e-16), a1[:, H:] / (a1[:, :H] + 1e-16)],
      axis=1)
  t = jnp.dot(h, fw_ref[...], preferred_element_type=F32) + fb_ref[...]
  ps_h = jnp.dot(t, wsh_ref[0], preferred_element_type=F32)
  t_h = jnp.dot(h, fwh_ref[0], preferred_element_type=F32) + fbh_ref[0]
  src2_ref[0] = jnp.concatenate([ps_h, t_h], axis=1)
  pd_ref[...] = jnp.dot(t, wd_ref[...], preferred_element_type=F32)


_k2_mid = pl.pallas_call(
    _k2_mid_body,
    grid=(_NB, NC),
    in_specs=[
        pl.BlockSpec((NC, _RB, D), lambda i, c: (0, i, 0)),
        pl.BlockSpec((D, D), lambda i, c: (0, 0)),
        pl.BlockSpec((1, D), lambda i, c: (0, 0)),
        pl.BlockSpec((1, D, H), lambda i, c: (c, 0, 0)),
        pl.BlockSpec((1, 1, H), lambda i, c: (c, 0, 0)),
        pl.BlockSpec((D, D), lambda i, c: (0, 0)),
        pl.BlockSpec((1, D, H), lambda i, c: (c, 0, 0)),
    ],
    out_specs=[
        pl.BlockSpec((1, _RB, D), lambda i, c: (c, i, 0)),
        pl.BlockSpec((_RB, D), lambda i, c: (i, 0)),
    ],
    out_shape=[
        jax.ShapeDtypeStruct((NC, N, D), F32),
        jax.ShapeDtypeStruct((N, D), F32),
    ],
)


_EB = 2048
_NEB = E_PAD // _EB


def _k3_body(ea_ref, we_ref, be_ref, wa_ref, ab_ref, pe_ref):
  i = pl.program_id(0)
  m = jnp.dot(we_ref[...], wa_ref[...], preferred_element_type=F32)
  cvec = (jnp.dot(be_ref[...], wa_ref[...], preferred_element_type=F32)
          + ab_ref[...])
  pe = jnp.dot(ea_ref[...], m, preferred_element_type=F32) + cvec
  row = i * _EB + lax.broadcasted_iota(jnp.int32, (_EB, 1), 0)
  pe_ref[...] = jnp.where(row < E, pe, -1e30)


_k3 = pl.pallas_call(
    _k3_body,
    grid=(_NEB,),
    in_specs=[
        pl.BlockSpec((_EB, DE), lambda i: (i, 0)),
        pl.BlockSpec((DE, DE), lambda i: (0, 0)),
        pl.BlockSpec((1, DE), lambda i: (0, 0)),
        pl.BlockSpec((DE, D), lambda i: (0, 0)),
        pl.BlockSpec((1, D), lambda i: (0, 0)),
    ],
    out_specs=pl.BlockSpec((_EB, D), lambda i: (i, 0)),
    out_shape=jax.ShapeDtypeStruct((E_PAD, D), F32),
)


def _k4_body(acc_ref, out_ref):
  a0, a1 = acc_ref[0], acc_ref[1]
  out_ref[...] = jnp.concatenate(
      [a0[:, H:] / (a0[:, :H] + 1e-16), a1[:, H:] / (a1[:, :H] + 1e-16)],
      axis=1)


_k4 = pl.pallas_call(
    _k4_body,
    grid=(_NB,),
    in_specs=[
        pl.BlockSpec((NC, _RB, D), lambda i: (0, i, 0)),
    ],
    out_specs=pl.BlockSpec((_RB, D), lambda i: (i, 0)),
    out_shape=jax.ShapeDtypeStruct((N, D), F32),
)


@functools.cache
def _build_sc_edge_pass():
  sc_mesh = plsc.VectorSubcoreMesh(
      core_axis_name="c", subcore_axis_name="s", num_cores=NC,
      num_subcores=NS)

  @functools.partial(
      pl.kernel,
      out_type=jax.ShapeDtypeStruct((NC * N_ACC, D), F32),
      mesh=sc_mesh,
      scratch_types=[
          pltpu.VMEM((CHUNK,), jnp.int32),
          pltpu.VMEM((CHUNK,), jnp.int32),
          pltpu.VMEM((CHUNK,), jnp.int32),
          pltpu.VMEM((CHUNK,), jnp.int32),
          pltpu.VMEM((CHUNK, D), F32),
          pltpu.VMEM((CHUNK, D), F32),
          pltpu.VMEM((CHUNK, D), F32),
          pltpu.VMEM((CHUNK, D), F32),
          pltpu.VMEM((CHUNK, D), F32),
          pltpu.VMEM((CHUNK, D), F32),
          pltpu.VMEM_SHARED((N_ACC, D), F32),
          pltpu.SemaphoreType.DMA,
          pltpu.SemaphoreType.DMA,
      ],
  )
  def sc_edge_pass(src_hbm, dst_hbm, src2_hbm, pd_hbm, pe_hbm,
                   acc_hbm,
                   isa, ida, isb, idb, sra, pda, pea, srb, pdb, peb,
                   acc, sema, semb):
    c = lax.axis_index("c")
    s = lax.axis_index("s")
    cH = c * H
    set_a = (isa, ida, sra, pda, pea, sema)
    set_b = (isb, idb, srb, pdb, peb, semb)

    zero = jnp.zeros((16,), F32)
    for r in range(8):
      for j in range(D // 16):
        sra[r, pl.ds(j * 16, 16)] = zero

    def zinit(k, carry):
      pltpu.sync_copy(sra.at[pl.ds(0, 8)],
                      acc.at[pl.ds(s * ROWS_PER_TILE + k * 8, 8)])
      return carry

    lax.fori_loop(0, ROWS_PER_TILE // 8, zinit, 0)
    plsc.subcore_barrier()

    def prefetch(g, bufs):
      i_s, i_d, sr, pd_, pe_, sem = bufs
      base = s * EPT + g * CHUNK
      pltpu.sync_copy(src_hbm.at[pl.ds(base, CHUNK)], i_s)
      pltpu.sync_copy(dst_hbm.at[pl.ds(base, CHUNK)], i_d)
      for j in range(CHUNK // 16):
        sl = pl.ds(j * 16, 16)
        i_s[sl] = i_s[sl] + c * N
      pltpu.async_copy(src2_hbm.at[i_s], sr, sem)
      pltpu.async_copy(pd_hbm.at[i_d], pd_, sem)
      pltpu.async_copy(pe_hbm.at[pl.ds(base, CHUNK)], pe_, sem)

    def wait_set(bufs):
      i_s, i_d, sr, pd_, pe_, sem = bufs
      pltpu.make_async_copy(src2_hbm.at[i_s], sr, sem).wait()
      pltpu.make_async_copy(pd_hbm.at[i_d], pd_, sem).wait()
      pltpu.make_async_copy(pe_hbm.at[pl.ds(0, CHUNK)], pe_, sem).wait()

    def compute_scatter(bufs):
      i_s, i_d, sr, pd_, pe_, sem = bufs

      @plsc.parallel_loop(0, CHUNK, step=1, unroll=4)
      def edge_body(e):
        for j in range(H // 16):
          slh = pl.ds(cH + j * 16, 16)
          sl = pl.ds(j * 16, 16)
          slt = pl.ds(H + j * 16, 16)
          a = pd_[e, slh] + sr[e, sl] + pe_[e, slh]
          a = jnp.maximum(a, a * 0.2)
          w = jnp.exp(a)
          tv = sr[e, slt]
          pd_[e, sl] = w
          pd_[e, slt] = w * tv
      pltpu.sync_copy(pd_, acc.at[i_d], add=True)

    def pair_body(i, carry):
      g0 = 2 * i
      wait_set(set_a)
      prefetch(g0 + 1, set_b)
      compute_scatter(set_a)
      wait_set(set_b)

      @pl.when(i + 1 < NSTEP // 2)
      def _():
        prefetch(g0 + 2, set_a)

      compute_scatter(set_b)
      return carry

    prefetch(0, set_a)
    lax.fori_loop(0, NSTEP // 2, pair_body, 0)
    plsc.subcore_barrier()

    def flush(k, carry):
      r0 = s * ROWS_PER_TILE + k * 64
      pltpu.sync_copy(acc.at[pl.ds(r0, 64)], peb)
      pltpu.sync_copy(peb, acc_hbm.at[pl.ds(c * N_ACC + r0, 64)])
      return carry

    lax.fori_loop(0, ROWS_PER_TILE // 64, flush, 0)
    tail = ROWS_PER_TILE % 64
    if tail:
      r0 = s * ROWS_PER_TILE + (ROWS_PER_TILE // 64) * 64
      pltpu.sync_copy(acc.at[pl.ds(r0, tail)], peb.at[pl.ds(0, tail)])
      pltpu.sync_copy(peb.at[pl.ds(0, tail)],
                      acc_hbm.at[pl.ds(c * N_ACC + r0, tail)])

  return sc_edge_pass


def _split_cols(w):
  return w.reshape(D, NC, H).transpose(1, 0, 2)


def kernel(x, edge_index, edge_attr, W_node, b_node, W_edge, b_edge,
           gamma, beta, fc_W, fc_b, attn_W, attn_b):
  src = edge_index[0].astype(jnp.int32)
  dst = edge_index[1].astype(jnp.int32)
  src_pad = jnp.pad(src, (0, E_PAD - E))
  dst_pad = jnp.pad(dst, (0, E_PAD - E))
  ea_pad = jnp.pad(edge_attr.astype(F32), ((0, E_PAD - E), (0, 0)))

  h_pre, psum, psumsq = _k1(x, W_node, b_node.reshape(1, D))

  acc = None
  for l in range(3):
    aW = attn_W[l]
    wd, ws, wa = aW[:D], aW[D:2 * D], aW[2 * D:]
    fw = fc_W[l]
    fwh = _split_cols(fw)
    fbh = fc_b[l].reshape(NC, 1, H)
    wsh = _split_cols(ws)
    if l == 0:
      src2, pd = _k2_first(h_pre, psum, psumsq, gamma.reshape(1, D),
                           beta.reshape(1, D), fw, fc_b[0].reshape(1, D),
                           fwh, fbh, wd, wsh)
    else:
      src2, pd = _k2_mid(acc, fw, fc_b[l].reshape(1, D), fwh, fbh, wd, wsh)
    pe = _k3(ea_pad, W_edge, b_edge.reshape(1, DE), wa,
             attn_b[l].reshape(1, D))
    acc = _build_sc_edge_pass()(
        src_pad, dst_pad, src2.reshape(NC * N, D), pd, pe)
    acc = acc.reshape(NC, N_ACC, D)[:, :N]

  h = _k4(acc)
  return h.reshape(1, N, D)


# speedup vs baseline: 4.6294x; 1.1468x over previous
import functools

import jax
import jax.numpy as jnp
from jax import lax
from jax.experimental import pallas as pl
from jax.experimental.pallas import tpu as pltpu
from jax.experimental.pallas import tpu_sc as plsc

N = 10000
E = 320000
D = 128
DE = 16
H = 64

NC = 2
NS = 16
CHUNK = 64
EPT = 20480
E_PAD = EPT * NS
NSTEP = EPT // CHUNK
N_ACC = 10112
ROWS_PER_TILE = N_ACC // NS

F32 = jnp.float32


_RB = 2000
_NB = N // _RB


def _k1_body(x_ref, w_ref, b_ref, h_ref, s_ref, ss_ref):
  h = jnp.dot(x_ref[...], w_ref[...], preferred_element_type=F32) + b_ref[...]
  h_ref[...] = h
  s_ref[...] = jnp.sum(h, axis=0, keepdims=True)[None]
  ss_ref[...] = jnp.sum(h * h, axis=0, keepdims=True)[None]


_k1 = pl.pallas_call(
    _k1_body,
    grid=(_NB,),
    in_specs=[
        pl.BlockSpec((_RB, D), lambda i: (i, 0)),
        pl.BlockSpec((D, D), lambda i: (0, 0)),
        pl.BlockSpec((1, D), lambda i: (0, 0)),
    ],
    out_specs=[
        pl.BlockSpec((_RB, D), lambda i: (i, 0)),
        pl.BlockSpec((1, 1, D), lambda i: (i, 0, 0)),
        pl.BlockSpec((1, 1, D), lambda i: (i, 0, 0)),
    ],
    out_shape=[
        jax.ShapeDtypeStruct((N, D), F32),
        jax.ShapeDtypeStruct((_NB, 1, D), F32),
        jax.ShapeDtypeStruct((_NB, 1, D), F32),
    ],
)


def _k2_first_body(h_ref, s_ref, ss_ref, g_ref, be_ref, fw_ref, fb_ref,
                   fwh_ref, fbh_ref, wd_ref, wsh_ref, src2_ref, pd_ref):
  mean = jnp.sum(s_ref[...], axis=0) / N
  var = jnp.sum(ss_ref[...], axis=0) / N - mean * mean
  hn = (h_ref[...] - mean) * (g_ref[...] * lax.rsqrt(var + 1e-5)) + be_ref[...]
  t = jnp.dot(hn, fw_ref[...], preferred_element_type=F32) + fb_ref[...]
  ps_h = jnp.dot(t, wsh_ref[0], preferred_element_type=F32)
  t_h = jnp.dot(hn, fwh_ref[0], preferred_element_type=F32) + fbh_ref[0]
  src2_ref[0] = jnp.concatenate([ps_h, t_h], axis=1)
  pd_ref[...] = jnp.dot(t, wd_ref[...], preferred_element_type=F32)


_k2_first = pl.pallas_call(
    _k2_first_body,
    grid=(_NB, NC),
    in_specs=[
        pl.BlockSpec((_RB, D), lambda i, c: (i, 0)),
        pl.BlockSpec((_NB, 1, D), lambda i, c: (0, 0, 0)),
        pl.BlockSpec((_NB, 1, D), lambda i, c: (0, 0, 0)),
        pl.BlockSpec((1, D), lambda i, c: (0, 0)),
        pl.BlockSpec((1, D), lambda i, c: (0, 0)),
        pl.BlockSpec((D, D), lambda i, c: (0, 0)),
        pl.BlockSpec((1, D), lambda i, c: (0, 0)),
        pl.BlockSpec((1, D, H), lambda i, c: (c, 0, 0)),
        pl.BlockSpec((1, 1, H), lambda i, c: (c, 0, 0)),
        pl.BlockSpec((D, D), lambda i, c: (0, 0)),
        pl.BlockSpec((1, D, H), lambda i, c: (c, 0, 0)),
    ],
    out_specs=[
        pl.BlockSpec((1, _RB, D), lambda i, c: (c, i, 0)),
        pl.BlockSpec((_RB, D), lambda i, c: (i, 0)),
    ],
    out_shape=[
        jax.ShapeDtypeStruct((NC, N, D), F32),
        jax.ShapeDtypeStruct((N, D), F32),
    ],
)


def _k2_mid_body(acc_ref, fw_ref, fb_ref, fwh_ref, fbh_ref, wd_ref, wsh_ref,
                 src2_ref, pd_ref):
  a0, a1 = acc_ref[0], acc_ref[1]
  h = jnp.concatenate(
      [a0[:, H:] / (a0[:, :H] + 1e-16), a1[:, H:] / (a1[:, :H] + 1e-16)],
      axis=1)
  t = jnp.dot(h, fw_ref[...], preferred_element_type=F32) + fb_ref[...]
  ps_h = jnp.dot(t, wsh_ref[0], preferred_element_type=F32)
  t_h = jnp.dot(h, fwh_ref[0], preferred_element_type=F32) + fbh_ref[0]
  src2_ref[0] = jnp.concatenate([ps_h, t_h], axis=1)
  pd_ref[...] = jnp.dot(t, wd_ref[...], preferred_element_type=F32)


_k2_mid = pl.pallas_call(
    _k2_mid_body,
    grid=(_NB, NC),
    in_specs=[
        pl.BlockSpec((NC, _RB, D), lambda i, c: (0, i, 0)),
        pl.BlockSpec((D, D), lambda i, c: (0, 0)),
        pl.BlockSpec((1, D), lambda i, c: (0, 0)),
        pl.BlockSpec((1, D, H), lambda i, c: (c, 0, 0)),
        pl.BlockSpec((1, 1, H), lambda i, c: (c, 0, 0)),
        pl.BlockSpec((D, D), lambda i, c: (0, 0)),
        pl.BlockSpec((1, D, H), lambda i, c: (c, 0, 0)),
    ],
    out_specs=[
        pl.BlockSpec((1, _RB, D), lambda i, c: (c, i, 0)),
        pl.BlockSpec((_RB, D), lambda i, c: (i, 0)),
    ],
    out_shape=[
        jax.ShapeDtypeStruct((NC, N, D), F32),
        jax.ShapeDtypeStruct((N, D), F32),
    ],
)


_EB = 2048
_NEB = E_PAD // _EB


def _k3_body(ea_ref, we_ref, be_ref, wa_ref, ab_ref, pe_ref):
  i = pl.program_id(0)
  m = jnp.dot(we_ref[...], wa_ref[...], preferred_element_type=F32)
  cvec = (jnp.dot(be_ref[...], wa_ref[...], preferred_element_type=F32)
          + ab_ref[...])
  pe = jnp.dot(ea_ref[...], m, preferred_element_type=F32) + cvec
  row = i * _EB + lax.broadcasted_iota(jnp.int32, (_EB, 1), 0)
  pe_ref[...] = jnp.where(row < E, pe, -1e30)


_k3 = pl.pallas_call(
    _k3_body,
    grid=(_NEB,),
    in_specs=[
        pl.BlockSpec((_EB, DE), lambda i: (i, 0)),
        pl.BlockSpec((DE, DE), lambda i: (0, 0)),
        pl.BlockSpec((1, DE), lambda i: (0, 0)),
        pl.BlockSpec((DE, D), lambda i: (0, 0)),
        pl.BlockSpec((1, D), lambda i: (0, 0)),
    ],
    out_specs=pl.BlockSpec((_EB, D), lambda i: (i, 0)),
    out_shape=jax.ShapeDtypeStruct((E_PAD, D), F32),
)


def _k4_body(acc_ref, out_ref):
  a0, a1 = acc_ref[0], acc_ref[1]
  out_ref[...] = jnp.concatenate(
      [a0[:, H:] / (a0[:, :H] + 1e-16), a1[:, H:] / (a1[:, :H] + 1e-16)],
      axis=1)


_k4 = pl.pallas_call(
    _k4_body,
    grid=(_NB,),
    in_specs=[
        pl.BlockSpec((NC, _RB, D), lambda i: (0, i, 0)),
    ],
    out_specs=pl.BlockSpec((_RB, D), lambda i: (i, 0)),
    out_shape=jax.ShapeDtypeStruct((N, D), F32),
)


@functools.cache
def _build_sc_edge_pass():
  sc_mesh = plsc.VectorSubcoreMesh(
      core_axis_name="c", subcore_axis_name="s", num_cores=NC,
      num_subcores=NS)

  @functools.partial(
      pl.kernel,
      out_type=jax.ShapeDtypeStruct((NC * N_ACC, D), F32),
      mesh=sc_mesh,
      scratch_types=[
          pltpu.VMEM((CHUNK,), jnp.int32),
          pltpu.VMEM((CHUNK,), jnp.int32),
          pltpu.VMEM((CHUNK,), jnp.int32),
          pltpu.VMEM((CHUNK,), jnp.int32),
          pltpu.VMEM((CHUNK, D), F32),
          pltpu.VMEM((CHUNK, D), F32),
          pltpu.VMEM((CHUNK, D), F32),
          pltpu.VMEM((CHUNK, D), F32),
          pltpu.VMEM((CHUNK, D), F32),
          pltpu.VMEM((CHUNK, D), F32),
          pltpu.VMEM_SHARED((N_ACC, D), F32),
          pltpu.SemaphoreType.DMA,
          pltpu.SemaphoreType.DMA,
          pltpu.SemaphoreType.DMA,
          pltpu.SemaphoreType.DMA,
      ],
  )
  def sc_edge_pass(src_hbm, dst_hbm, src2_hbm, pd_hbm, pe_hbm,
                   acc_hbm,
                   isa, ida, isb, idb, sra, pda, pea, srb, pdb, peb,
                   acc, sema, semb, isema, isemb):
    c = lax.axis_index("c")
    s = lax.axis_index("s")
    cH = c * H
    set_a = (isa, ida, sra, pda, pea, sema, isema)
    set_b = (isb, idb, srb, pdb, peb, semb, isemb)

    zero = jnp.zeros((16,), F32)
    for r in range(8):
      for j in range(D // 16):
        sra[r, pl.ds(j * 16, 16)] = zero

    def zinit(k, carry):
      pltpu.sync_copy(sra.at[pl.ds(0, 8)],
                      acc.at[pl.ds(s * ROWS_PER_TILE + k * 8, 8)])
      return carry

    lax.fori_loop(0, ROWS_PER_TILE // 8, zinit, 0)
    plsc.subcore_barrier()

    def issue_idx(g, bufs):
      i_s, i_d, _, _, _, _, isem = bufs
      base = s * EPT + g * CHUNK
      pltpu.async_copy(src_hbm.at[pl.ds(base, CHUNK)], i_s, isem)
      pltpu.async_copy(dst_hbm.at[pl.ds(base, CHUNK)], i_d, isem)

    def prefetch(g, bufs):
      i_s, i_d, sr, pd_, pe_, sem, isem = bufs
      base = s * EPT + g * CHUNK
      pltpu.make_async_copy(src_hbm.at[pl.ds(0, CHUNK)], i_s, isem).wait()
      pltpu.make_async_copy(dst_hbm.at[pl.ds(0, CHUNK)], i_d, isem).wait()
      for j in range(CHUNK // 16):
        sl = pl.ds(j * 16, 16)
        i_s[sl] = i_s[sl] + c * N
      pltpu.async_copy(src2_hbm.at[i_s], sr, sem)
      pltpu.async_copy(pd_hbm.at[i_d], pd_, sem)
      pltpu.async_copy(pe_hbm.at[pl.ds(base, CHUNK)], pe_, sem)

    def wait_set(bufs):
      i_s, i_d, sr, pd_, pe_, sem, _ = bufs
      pltpu.make_async_copy(src2_hbm.at[i_s], sr, sem).wait()
      pltpu.make_async_copy(pd_hbm.at[i_d], pd_, sem).wait()
      pltpu.make_async_copy(pe_hbm.at[pl.ds(0, CHUNK)], pe_, sem).wait()

    def compute_scatter(bufs):
      i_s, i_d, sr, pd_, pe_, sem, _ = bufs

      @plsc.parallel_loop(0, CHUNK, step=1, unroll=4)
      def edge_body(e):
        for j in range(H // 16):
          slh = pl.ds(cH + j * 16, 16)
          sl = pl.ds(j * 16, 16)
          slt = pl.ds(H + j * 16, 16)
          a = pd_[e, slh] + sr[e, sl] + pe_[e, slh]
          a = jnp.maximum(a, a * 0.2)
          w = jnp.exp(a)
          tv = sr[e, slt]
          pd_[e, sl] = w
          pd_[e, slt] = w * tv
      pltpu.sync_copy(pd_, acc.at[i_d], add=True)

    def pair_body(i, carry):
      g0 = 2 * i
      wait_set(set_a)
      prefetch(g0 + 1, set_b)
      compute_scatter(set_a)

      @pl.when(g0 + 2 < NSTEP)
      def _():
        issue_idx(g0 + 2, set_a)

      wait_set(set_b)

      @pl.when(g0 + 2 < NSTEP)
      def _():
        prefetch(g0 + 2, set_a)

      compute_scatter(set_b)

      @pl.when(g0 + 3 < NSTEP)
      def _():
        issue_idx(g0 + 3, set_b)

      return carry

    issue_idx(0, set_a)
    prefetch(0, set_a)
    issue_idx(1, set_b)
    lax.fori_loop(0, NSTEP // 2, pair_body, 0)
    plsc.subcore_barrier()

    def flush(k, carry):
      r0 = s * ROWS_PER_TILE + k * 64
      pltpu.sync_copy(acc.at[pl.ds(r0, 64)], peb)
      pltpu.sync_copy(peb, acc_hbm.at[pl.ds(c * N_ACC + r0, 64)])
      return carry

    lax.fori_loop(0, ROWS_PER_TILE // 64, flush, 0)
    tail = ROWS_PER_TILE % 64
    if tail:
      r0 = s * ROWS_PER_TILE + (ROWS_PER_TILE // 64) * 64
      pltpu.sync_copy(acc.at[pl.ds(r0, tail)], peb.at[pl.ds(0, tail)])
      pltpu.sync_copy(peb.at[pl.ds(0, tail)],
                      acc_hbm.at[pl.ds(c * N_ACC + r0, tail)])

  return sc_edge_pass


def _split_cols(w):
  return w.reshape(D, NC, H).transpose(1, 0, 2)


def kernel(x, edge_index, edge_attr, W_node, b_node, W_edge, b_edge,
           gamma, beta, fc_W, fc_b, attn_W, attn_b):
  src = edge_index[0].astype(jnp.int32)
  dst = edge_index[1].astype(jnp.int32)
  src_pad = jnp.pad(src, (0, E_PAD - E))
  dst_pad = jnp.pad(dst, (0, E_PAD - E))
  ea_pad = jnp.pad(edge_attr.astype(F32), ((0, E_PAD - E), (0, 0)))

  h_pre, psum, psumsq = _k1(x, W_node, b_node.reshape(1, D))

  acc = None
  for l in range(3):
    aW = attn_W[l]
    wd, ws, wa = aW[:D], aW[D:2 * D], aW[2 * D:]
    fw = fc_W[l]
    fwh = _split_cols(fw)
    fbh = fc_b[l].reshape(NC, 1, H)
    wsh = _split_cols(ws)
    if l == 0:
      src2, pd = _k2_first(h_pre, psum, psumsq, gamma.reshape(1, D),
                           beta.reshape(1, D), fw, fc_b[0].reshape(1, D),
                           fwh, fbh, wd, wsh)
    else:
      src2, pd = _k2_mid(acc, fw, fc_b[l].reshape(1, D), fwh, fbh, wd, wsh)
    pe = _k3(ea_pad, W_edge, b_edge.reshape(1, DE), wa,
             attn_b[l].reshape(1, D))
    acc = _build_sc_edge_pass()(
        src_pad, dst_pad, src2.reshape(NC * N, D), pd, pe)
    acc = acc.reshape(NC, N_ACC, D)[:, :N]

  h = _k4(acc)
  return h.reshape(1, N, D)
